# baseline (speedup 1.0000x reference)
import functools
import jax
import jax.numpy as jnp
from jax import lax
from jax.experimental import pallas as pl
from jax.experimental.pallas import tpu as pltpu
from jax.experimental.pallas import tpu_sc as plsc

N_NODES = 100000
NDEG = 100352
DEG_SLICE = NDEG // 16


BN = 2048


def _tc_a1_body(x_ref, wtT_ref, bt_ref, wgT_ref, xw_ref):
    xv = x_ref[...]
    h = jnp.dot(xv, wtT_ref[...], preferred_element_type=jnp.float32)
    h = jnp.maximum(h + bt_ref[...][None, :], 0.0)
    xw_ref[...] = jnp.dot(h, wgT_ref[...],
                          preferred_element_type=jnp.float32)


def _tc_a2_body(xw_ref, d0_ref, d1_ref, xws_ref, dinv_ref):
    deg = d0_ref[...] + d1_ref[...] + 1.0
    dinv = jnp.where(deg > 0, jax.lax.rsqrt(deg), 0.0)
    dinv_ref[...] = dinv
    xws_ref[...] = xw_ref[...] * dinv[:, None]


def _tc_b_body(acc_ref, xws_ref, dinv_ref, bg_ref, whT_ref, bh_ref, out_ref):
    dinv = dinv_ref[...]
    h2 = dinv[:, None] * (acc_ref[...] + xws_ref[...]) + bg_ref[...][None, :]
    h2 = jnp.maximum(h2, 0.0)
    out_ref[...] = (jnp.sum(h2 * whT_ref[...], axis=1, keepdims=True)
                    + bh_ref[0])


def _sc_deg_body(col2d, ew2d, degp, deg_sp, colv, ewv, zv, sem):
    c = lax.axis_index("c")
    t = lax.axis_index("s")
    nrows = col2d.shape[0]
    rows_per_core = nrows // 2
    rows_per_tile = rows_per_core // 16
    nchunks = rows_per_tile // 8

    def _z(i, _):
        zv[pl.ds(i * 16, 16)] = jnp.zeros((16,), jnp.float32)
        return 0
    lax.fori_loop(0, DEG_SLICE // 16, _z, 0)
    pltpu.sync_copy(zv, deg_sp.at[pl.ds(t * DEG_SLICE, DEG_SLICE)])
    plsc.subcore_barrier()

    row_base = c * rows_per_core + t * rows_per_tile

    def _chunk(k, _):
        r0 = row_base + k * 8
        pltpu.sync_copy(col2d.at[pl.ds(r0, 8)], colv)
        pltpu.sync_copy(ew2d.at[pl.ds(r0, 8)], ewv)
        descs = []
        for j in range(8):
            descs.append(pltpu.async_copy(
                ewv.at[j], deg_sp.at[colv.at[j]], sem, add=True))
        for d in descs:
            d.wait()
        return 0
    lax.fori_loop(0, nchunks, _chunk, 0)

    plsc.subcore_barrier()
    pltpu.sync_copy(deg_sp.at[pl.ds(t * DEG_SLICE, DEG_SLICE)],
                    degp.at[c].at[pl.ds(t * DEG_SLICE, DEG_SLICE)])


def _sc_deg(col2d, ew2d):
    mesh = plsc.VectorSubcoreMesh(core_axis_name="c", subcore_axis_name="s")
    f = pl.kernel(
        _sc_deg_body,
        out_type=jax.ShapeDtypeStruct((2, NDEG), jnp.float32),
        mesh=mesh,
        scratch_types=[
            pltpu.VMEM_SHARED((NDEG,), jnp.float32),
            pltpu.VMEM((8, 128), jnp.int32),
            pltpu.VMEM((8, 128), jnp.float32),
            pltpu.VMEM((DEG_SLICE,), jnp.float32),
            pltpu.SemaphoreType.DMA,
        ],
    )
    return f(col2d, ew2d)


HALF = 50000
ACC_TILE = 3136
ACC_ROWS = 50304
DUMMY = 50176
OUT_TILE = 3128
OUT_LAST = 50000 - 15 * OUT_TILE


def _sc_acc_body(row2d, col2d, ew2d, xws, acc_out,
                 acc_sp, eb0, eb1, cl0, cl1, wb0, wb1, ix0, ix1,
                 rv0, rv1, rv2, rv3,
                 zb, ls0, ls1, gs0, gs1, gs2, gs3, ss0, ss1, ss2, ss3):
    c = lax.axis_index("c")
    t = lax.axis_index("s")
    base = c * HALF
    nrows = row2d.shape[0]
    rows_per_tile = nrows // 16
    row0 = t * rows_per_tile

    def _z(i, _):
        zb[i, pl.ds(0, 16)] = jnp.zeros((16,), jnp.float32)
        zb[i, pl.ds(16, 16)] = jnp.zeros((16,), jnp.float32)
        return 0
    lax.fori_loop(0, zb.shape[0], _z, 0)
    for m in range(ACC_TILE // 98):
        pltpu.sync_copy(zb, acc_sp.at[pl.ds(t * ACC_TILE + m * 98, 98), :])
    plsc.subcore_barrier()

    ebufs = (eb0, eb1)
    clbufs = (cl0, cl1)
    wbufs = (wb0, wb1)
    ixbufs = (ix0, ix1)
    rvs = (rv0, rv1, rv2, rv3)
    lsems = (ls0, ls1)
    gsems = (gs0, gs1, gs2, gs3)
    ssems = (ss0, ss1, ss2, ss3)
    NG = rows_per_tile // 4

    def _stage(g, p):
        r0 = row0 + g * 4
        pltpu.async_copy(row2d.at[pl.ds(r0, 4)], ebufs[p], lsems[p])
        pltpu.async_copy(col2d.at[pl.ds(r0, 4)], clbufs[p], lsems[p])
        pltpu.async_copy(ew2d.at[pl.ds(r0, 4)], wbufs[p], lsems[p])

    def _wait_stage(p):
        pltpu.make_async_copy(row2d.at[pl.ds(row0, 4)],
                              ebufs[p], lsems[p]).wait()
        pltpu.make_async_copy(col2d.at[pl.ds(row0, 4)],
                              clbufs[p], lsems[p]).wait()
        pltpu.make_async_copy(ew2d.at[pl.ds(row0, 4)],
                              wbufs[p], lsems[p]).wait()

    def _fire_gather(p, j, b):
        pltpu.async_copy(xws.at[ebufs[p].at[j]], rvs[b], gsems[b])

    def _drain_scatter(b, p, j):
        pltpu.make_async_copy(rvs[b], acc_sp.at[ixbufs[p].at[j]],
                              ssems[b]).wait()

    def _group(g, p):
        @pl.when(g < NG - 1)
        def _prefetch():
            _stage(g + 1, 1 - p)

        eb = ebufs[p]
        ix = ixbufs[p]
        for j in range(4):
            jp = (j + 3) % 4
            pltpu.make_async_copy(xws.at[eb.at[j]],
                                  rvs[j], gsems[j]).wait()
            if j >= 1:
                _drain_scatter(jp, p, j - 1)
            else:
                @pl.when(g >= 1)
                def _dr():
                    _drain_scatter(3, 1 - p, 3)
            if j == 0:
                _fire_gather(p, 3, 3)
            else:
                @pl.when(g < NG - 1)
                def _cross():
                    if j == 1:
                        _wait_stage(1 - p)
                    _fire_gather(1 - p, j - 1, j - 1)

            def _scale(k, _):
                q0 = pl.multiple_of(k * 16, 16)
                colg = clbufs[p][j, pl.ds(q0, 16)]
                ewg = wbufs[p][j, pl.ds(q0, 16)]
                tgt = colg - base
                valid = (tgt >= 0) & (tgt < HALF)
                ew_eff = jnp.where(valid, ewg, 0.0)
                spread = DUMMY + q0 + lax.iota(jnp.int32, 16)
                idxg = jnp.where(valid, tgt, spread)
                ix[j, pl.ds(q0, 16)] = idxg
                for u in range(16):
                    s_u = lax.squeeze(lax.slice(ew_eff, (u,), (u + 1,)),
                                      (0,))
                    rvs[j][q0 + u, pl.ds(0, 16)] = (
                        rvs[j][q0 + u, pl.ds(0, 16)] * s_u)
                    rvs[j][q0 + u, pl.ds(16, 16)] = (
                        rvs[j][q0 + u, pl.ds(16, 16)] * s_u)
                return 0
            lax.fori_loop(0, 8, _scale, 0)

            pltpu.async_copy(rvs[j], acc_sp.at[ix.at[j]], ssems[j],
                             add=True)

    _stage(0, 0)
    _wait_stage(0)
    _fire_gather(0, 0, 0)
    _fire_gather(0, 1, 1)
    _fire_gather(0, 2, 2)

    def _pair(h, _):
        _group(2 * h, 0)
        _group(2 * h + 1, 1)
        return 0
    lax.fori_loop(0, NG // 2, _pair, 0)

    _drain_scatter(3, 1, 3)

    plsc.subcore_barrier()

    @pl.when(t < 15)
    def _copy_main():
        pltpu.sync_copy(acc_sp.at[pl.ds(t * OUT_TILE, OUT_TILE), :],
                        acc_out.at[pl.ds(base + t * OUT_TILE, OUT_TILE), :])

    @pl.when(t == 15)
    def _copy_last():
        pltpu.sync_copy(acc_sp.at[pl.ds(15 * OUT_TILE, OUT_LAST), :],
                        acc_out.at[pl.ds(base + 15 * OUT_TILE, OUT_LAST), :])


def _sc_acc(row2d, col2d, ew2d, xws):
    mesh = plsc.VectorSubcoreMesh(core_axis_name="c", subcore_axis_name="s")
    f = pl.kernel(
        _sc_acc_body,
        out_type=jax.ShapeDtypeStruct((N_NODES, 32), jnp.float32),
        mesh=mesh,
        scratch_types=[
            pltpu.VMEM_SHARED((ACC_ROWS, 32), jnp.float32),
            pltpu.VMEM((4, 128), jnp.int32),
            pltpu.VMEM((4, 128), jnp.int32),
            pltpu.VMEM((4, 128), jnp.int32),
            pltpu.VMEM((4, 128), jnp.int32),
            pltpu.VMEM((4, 128), jnp.float32),
            pltpu.VMEM((4, 128), jnp.float32),
            pltpu.VMEM((4, 128), jnp.int32),
            pltpu.VMEM((4, 128), jnp.int32),
            pltpu.VMEM((128, 32), jnp.float32),
            pltpu.VMEM((128, 32), jnp.float32),
            pltpu.VMEM((128, 32), jnp.float32),
            pltpu.VMEM((128, 32), jnp.float32),
            pltpu.VMEM((98, 32), jnp.float32),
            pltpu.SemaphoreType.DMA,
            pltpu.SemaphoreType.DMA,
            pltpu.SemaphoreType.DMA,
            pltpu.SemaphoreType.DMA,
            pltpu.SemaphoreType.DMA,
            pltpu.SemaphoreType.DMA,
            pltpu.SemaphoreType.DMA,
            pltpu.SemaphoreType.DMA,
            pltpu.SemaphoreType.DMA,
            pltpu.SemaphoreType.DMA,
        ],
        compiler_params=pltpu.CompilerParams(use_tc_tiling_on_sc=False),
    )
    return f(row2d, col2d, ew2d, xws)


def _full1d(shape):
    return pl.BlockSpec(shape, lambda i: tuple(0 for _ in shape))


def _tc_a1(xv, wtT, bt, wgT):
    n = xv.shape[0]
    grid = pl.cdiv(n, BN)
    return pl.pallas_call(
        _tc_a1_body,
        grid=(grid,),
        in_specs=[
            pl.BlockSpec((BN, xv.shape[1]), lambda i: (i, 0)),
            _full1d(wtT.shape),
            _full1d(bt.shape),
            _full1d(wgT.shape),
        ],
        out_specs=pl.BlockSpec((BN, 32), lambda i: (i, 0)),
        out_shape=jax.ShapeDtypeStruct((n, 32), jnp.float32),
    )(xv, wtT, bt, wgT)


def _tc_a2(xw, d0, d1):
    n = xw.shape[0]
    grid = pl.cdiv(n, BN)
    return pl.pallas_call(
        _tc_a2_body,
        grid=(grid,),
        in_specs=[
            pl.BlockSpec((BN, 32), lambda i: (i, 0)),
            pl.BlockSpec((BN,), lambda i: (i,)),
            pl.BlockSpec((BN,), lambda i: (i,)),
        ],
        out_specs=[
            pl.BlockSpec((BN, 32), lambda i: (i, 0)),
            pl.BlockSpec((BN,), lambda i: (i,)),
        ],
        out_shape=[
            jax.ShapeDtypeStruct((n, 32), jnp.float32),
            jax.ShapeDtypeStruct((n,), jnp.float32),
        ],
    )(xw, d0, d1)


def _tc_b(acc, xws, dinv, bg, whT, bh):
    n = acc.shape[0]
    grid = pl.cdiv(n, BN)
    return pl.pallas_call(
        _tc_b_body,
        grid=(grid,),
        in_specs=[
            pl.BlockSpec((BN, 32), lambda i: (i, 0)),
            pl.BlockSpec((BN, 32), lambda i: (i, 0)),
            pl.BlockSpec((BN,), lambda i: (i,)),
            _full1d(bg.shape),
            _full1d(whT.shape),
            _full1d(bh.shape),
        ],
        out_specs=pl.BlockSpec((BN, 1), lambda i: (i, 0)),
        out_shape=jax.ShapeDtypeStruct((n, 1), jnp.float32),
    )(acc, xws, dinv, bg, whT, bh)


@jax.jit
def kernel(x, edge_index, edge_weight, Wt, bt, Wg, bg, Wh, bh):
    n = x.shape[0]
    xv = x.reshape(n, -1)
    wtT = Wt.reshape(Wt.shape[0], -1).T
    wgT = Wg.T
    whT = Wh.T
    row = edge_index[0]
    col = edge_index[1]

    e = row.shape[0]
    epad = ((e + 32767) // 32768) * 32768
    padn = epad - e
    rowp = jnp.concatenate([row, jnp.zeros((padn,), row.dtype)])
    colp = jnp.concatenate([col, jnp.full((padn,), n, col.dtype)])
    ewp = jnp.concatenate([edge_weight,
                           jnp.zeros((padn,), edge_weight.dtype)])
    col2d = colp.reshape(-1, 128)
    ew2d = ewp.reshape(-1, 128)

    degp = _sc_deg(col2d, ew2d)
    d0 = degp[0, :n]
    d1 = degp[1, :n]

    xw = _tc_a1(xv, wtT, bt, wgT)
    xws, dinv = _tc_a2(xw, d0, d1)

    acc = _sc_acc(rowp.reshape(-1, 128), col2d, ew2d, xws)

    return _tc_b(acc, xws, dinv, bg, whT, bh)


# speedup vs baseline: 1.0132x; 1.0132x over previous
import functools
import jax
import jax.numpy as jnp
from jax import lax
from jax.experimental import pallas as pl
from jax.experimental.pallas import tpu as pltpu
from jax.experimental.pallas import tpu_sc as plsc

N_NODES = 100000
NDEG = 100352
DEG_SLICE = NDEG // 16


BN = 2048


def _tc_a_body(x_ref, d0_ref, d1_ref, wtT_ref, bt_ref, wgT_ref,
               xws_ref, dinv_ref):
    xv = x_ref[...]
    h = jnp.dot(xv, wtT_ref[...], preferred_element_type=jnp.float32)
    h = jnp.maximum(h + bt_ref[...][None, :], 0.0)
    xw = jnp.dot(h, wgT_ref[...], preferred_element_type=jnp.float32)
    deg = d0_ref[...] + d1_ref[...] + 1.0
    dinv = jnp.where(deg > 0, jax.lax.rsqrt(deg), 0.0)
    dinv_ref[...] = dinv
    xws_ref[...] = xw * dinv[:, None]


def _tc_b_body(acc_ref, xws_ref, dinv_ref, bg_ref, whT_ref, bh_ref, out_ref):
    dinv = dinv_ref[...]
    h2 = dinv[:, None] * (acc_ref[...] + xws_ref[...]) + bg_ref[...][None, :]
    h2 = jnp.maximum(h2, 0.0)
    out_ref[...] = (jnp.sum(h2 * whT_ref[...], axis=1, keepdims=True)
                    + bh_ref[0])


def _sc_deg_body(col2d, ew2d, degp, deg_sp, colv, ewv, zv, sem):
    c = lax.axis_index("c")
    t = lax.axis_index("s")
    nrows = col2d.shape[0]
    rows_per_core = nrows // 2
    rows_per_tile = rows_per_core // 16
    nchunks = rows_per_tile // 8

    def _z(i, _):
        zv[pl.ds(i * 16, 16)] = jnp.zeros((16,), jnp.float32)
        return 0
    lax.fori_loop(0, DEG_SLICE // 16, _z, 0)
    pltpu.sync_copy(zv, deg_sp.at[pl.ds(t * DEG_SLICE, DEG_SLICE)])
    plsc.subcore_barrier()

    row_base = c * rows_per_core + t * rows_per_tile

    def _chunk(k, _):
        r0 = row_base + k * 8
        pltpu.sync_copy(col2d.at[pl.ds(r0, 8)], colv)
        pltpu.sync_copy(ew2d.at[pl.ds(r0, 8)], ewv)
        descs = []
        for j in range(8):
            descs.append(pltpu.async_copy(
                ewv.at[j], deg_sp.at[colv.at[j]], sem, add=True))
        for d in descs:
            d.wait()
        return 0
    lax.fori_loop(0, nchunks, _chunk, 0)

    plsc.subcore_barrier()
    pltpu.sync_copy(deg_sp.at[pl.ds(t * DEG_SLICE, DEG_SLICE)],
                    degp.at[c].at[pl.ds(t * DEG_SLICE, DEG_SLICE)])


def _sc_deg(col2d, ew2d):
    mesh = plsc.VectorSubcoreMesh(core_axis_name="c", subcore_axis_name="s")
    f = pl.kernel(
        _sc_deg_body,
        out_type=jax.ShapeDtypeStruct((2, NDEG), jnp.float32),
        mesh=mesh,
        scratch_types=[
            pltpu.VMEM_SHARED((NDEG,), jnp.float32),
            pltpu.VMEM((8, 128), jnp.int32),
            pltpu.VMEM((8, 128), jnp.float32),
            pltpu.VMEM((DEG_SLICE,), jnp.float32),
            pltpu.SemaphoreType.DMA,
        ],
    )
    return f(col2d, ew2d)


HALF = 50000
ACC_TILE = 3136
ACC_ROWS = 50304
DUMMY = 50176
OUT_TILE = 3128
OUT_LAST = 50000 - 15 * OUT_TILE


def _sc_acc_body(row2d, col2d, ew2d, xws, acc_out,
                 acc_sp, eb0, eb1, cl0, cl1, wb0, wb1, ix0, ix1,
                 rv0, rv1, rv2, rv3,
                 zb, ls0, ls1, gs0, gs1, gs2, gs3, ss0, ss1, ss2, ss3):
    c = lax.axis_index("c")
    t = lax.axis_index("s")
    base = c * HALF
    nrows = row2d.shape[0]
    rows_per_tile = nrows // 16
    row0 = t * rows_per_tile

    def _z(i, _):
        zb[i, pl.ds(0, 16)] = jnp.zeros((16,), jnp.float32)
        zb[i, pl.ds(16, 16)] = jnp.zeros((16,), jnp.float32)
        return 0
    lax.fori_loop(0, zb.shape[0], _z, 0)
    for m in range(ACC_TILE // 98):
        pltpu.sync_copy(zb, acc_sp.at[pl.ds(t * ACC_TILE + m * 98, 98), :])
    plsc.subcore_barrier()

    ebufs = (eb0, eb1)
    clbufs = (cl0, cl1)
    wbufs = (wb0, wb1)
    ixbufs = (ix0, ix1)
    rvs = (rv0, rv1, rv2, rv3)
    lsems = (ls0, ls1)
    gsems = (gs0, gs1, gs2, gs3)
    ssems = (ss0, ss1, ss2, ss3)
    NG = rows_per_tile // 4

    def _stage(g, p):
        r0 = row0 + g * 4
        pltpu.async_copy(row2d.at[pl.ds(r0, 4)], ebufs[p], lsems[p])
        pltpu.async_copy(col2d.at[pl.ds(r0, 4)], clbufs[p], lsems[p])
        pltpu.async_copy(ew2d.at[pl.ds(r0, 4)], wbufs[p], lsems[p])

    def _wait_stage(p):
        pltpu.make_async_copy(row2d.at[pl.ds(row0, 4)],
                              ebufs[p], lsems[p]).wait()
        pltpu.make_async_copy(col2d.at[pl.ds(row0, 4)],
                              clbufs[p], lsems[p]).wait()
        pltpu.make_async_copy(ew2d.at[pl.ds(row0, 4)],
                              wbufs[p], lsems[p]).wait()

    def _fire_gather(p, j, b):
        pltpu.async_copy(xws.at[ebufs[p].at[j]], rvs[b], gsems[b])

    def _drain_scatter(b, p, j):
        pltpu.make_async_copy(rvs[b], acc_sp.at[ixbufs[p].at[j]],
                              ssems[b]).wait()

    def _group(g, p):
        @pl.when(g < NG - 1)
        def _prefetch():
            _stage(g + 1, 1 - p)

        eb = ebufs[p]
        ix = ixbufs[p]
        for j in range(4):
            jp = (j + 3) % 4
            pltpu.make_async_copy(xws.at[eb.at[j]],
                                  rvs[j], gsems[j]).wait()
            if j >= 1:
                _drain_scatter(jp, p, j - 1)
            else:
                @pl.when(g >= 1)
                def _dr():
                    _drain_scatter(3, 1 - p, 3)
            if j == 0:
                _fire_gather(p, 3, 3)
            else:
                @pl.when(g < NG - 1)
                def _cross():
                    if j == 1:
                        _wait_stage(1 - p)
                    _fire_gather(1 - p, j - 1, j - 1)

            def _scale(k, _):
                q0 = pl.multiple_of(k * 16, 16)
                colg = clbufs[p][j, pl.ds(q0, 16)]
                ewg = wbufs[p][j, pl.ds(q0, 16)]
                tgt = colg - base
                valid = (tgt >= 0) & (tgt < HALF)
                ew_eff = jnp.where(valid, ewg, 0.0)
                spread = DUMMY + q0 + lax.iota(jnp.int32, 16)
                idxg = jnp.where(valid, tgt, spread)
                ix[j, pl.ds(q0, 16)] = idxg
                for u in range(16):
                    s_u = lax.squeeze(lax.slice(ew_eff, (u,), (u + 1,)),
                                      (0,))
                    rvs[j][q0 + u, pl.ds(0, 16)] = (
                        rvs[j][q0 + u, pl.ds(0, 16)] * s_u)
                    rvs[j][q0 + u, pl.ds(16, 16)] = (
                        rvs[j][q0 + u, pl.ds(16, 16)] * s_u)
                return 0
            lax.fori_loop(0, 8, _scale, 0)

            pltpu.async_copy(rvs[j], acc_sp.at[ix.at[j]], ssems[j],
                             add=True)

    _stage(0, 0)
    _wait_stage(0)
    _fire_gather(0, 0, 0)
    _fire_gather(0, 1, 1)
    _fire_gather(0, 2, 2)

    def _pair(h, _):
        _group(2 * h, 0)
        _group(2 * h + 1, 1)
        return 0
    lax.fori_loop(0, NG // 2, _pair, 0)

    _drain_scatter(3, 1, 3)

    plsc.subcore_barrier()

    @pl.when(t < 15)
    def _copy_main():
        pltpu.sync_copy(acc_sp.at[pl.ds(t * OUT_TILE, OUT_TILE), :],
                        acc_out.at[pl.ds(base + t * OUT_TILE, OUT_TILE), :])

    @pl.when(t == 15)
    def _copy_last():
        pltpu.sync_copy(acc_sp.at[pl.ds(15 * OUT_TILE, OUT_LAST), :],
                        acc_out.at[pl.ds(base + 15 * OUT_TILE, OUT_LAST), :])


def _sc_acc(row2d, col2d, ew2d, xws):
    mesh = plsc.VectorSubcoreMesh(core_axis_name="c", subcore_axis_name="s")
    f = pl.kernel(
        _sc_acc_body,
        out_type=jax.ShapeDtypeStruct((N_NODES, 32), jnp.float32),
        mesh=mesh,
        scratch_types=[
            pltpu.VMEM_SHARED((ACC_ROWS, 32), jnp.float32),
            pltpu.VMEM((4, 128), jnp.int32),
            pltpu.VMEM((4, 128), jnp.int32),
            pltpu.VMEM((4, 128), jnp.int32),
            pltpu.VMEM((4, 128), jnp.int32),
            pltpu.VMEM((4, 128), jnp.float32),
            pltpu.VMEM((4, 128), jnp.float32),
            pltpu.VMEM((4, 128), jnp.int32),
            pltpu.VMEM((4, 128), jnp.int32),
            pltpu.VMEM((128, 32), jnp.float32),
            pltpu.VMEM((128, 32), jnp.float32),
            pltpu.VMEM((128, 32), jnp.float32),
            pltpu.VMEM((128, 32), jnp.float32),
            pltpu.VMEM((98, 32), jnp.float32),
            pltpu.SemaphoreType.DMA,
            pltpu.SemaphoreType.DMA,
            pltpu.SemaphoreType.DMA,
            pltpu.SemaphoreType.DMA,
            pltpu.SemaphoreType.DMA,
            pltpu.SemaphoreType.DMA,
            pltpu.SemaphoreType.DMA,
            pltpu.SemaphoreType.DMA,
            pltpu.SemaphoreType.DMA,
            pltpu.SemaphoreType.DMA,
        ],
        compiler_params=pltpu.CompilerParams(use_tc_tiling_on_sc=False),
    )
    return f(row2d, col2d, ew2d, xws)


def _full1d(shape):
    return pl.BlockSpec(shape, lambda i: tuple(0 for _ in shape))


def _tc_a(xv, d0, d1, wtT, bt, wgT):
    n = xv.shape[0]
    grid = pl.cdiv(n, BN)
    return pl.pallas_call(
        _tc_a_body,
        grid=(grid,),
        in_specs=[
            pl.BlockSpec((BN, xv.shape[1]), lambda i: (i, 0)),
            pl.BlockSpec((BN,), lambda i: (i,)),
            pl.BlockSpec((BN,), lambda i: (i,)),
            _full1d(wtT.shape),
            _full1d(bt.shape),
            _full1d(wgT.shape),
        ],
        out_specs=[
            pl.BlockSpec((BN, 32), lambda i: (i, 0)),
            pl.BlockSpec((BN,), lambda i: (i,)),
        ],
        out_shape=[
            jax.ShapeDtypeStruct((n, 32), jnp.float32),
            jax.ShapeDtypeStruct((n,), jnp.float32),
        ],
    )(xv, d0, d1, wtT, bt, wgT)


def _tc_b(acc, xws, dinv, bg, whT, bh):
    n = acc.shape[0]
    grid = pl.cdiv(n, BN)
    return pl.pallas_call(
        _tc_b_body,
        grid=(grid,),
        in_specs=[
            pl.BlockSpec((BN, 32), lambda i: (i, 0)),
            pl.BlockSpec((BN, 32), lambda i: (i, 0)),
            pl.BlockSpec((BN,), lambda i: (i,)),
            _full1d(bg.shape),
            _full1d(whT.shape),
            _full1d(bh.shape),
        ],
        out_specs=pl.BlockSpec((BN, 1), lambda i: (i, 0)),
        out_shape=jax.ShapeDtypeStruct((n, 1), jnp.float32),
    )(acc, xws, dinv, bg, whT, bh)


@jax.jit
def kernel(x, edge_index, edge_weight, Wt, bt, Wg, bg, Wh, bh):
    n = x.shape[0]
    xv = x.reshape(n, -1)
    wtT = Wt.reshape(Wt.shape[0], -1).T
    wgT = Wg.T
    whT = Wh.T
    row = edge_index[0]
    col = edge_index[1]

    e = row.shape[0]
    epad = ((e + 32767) // 32768) * 32768
    padn = epad - e
    rowp = jnp.concatenate([row, jnp.zeros((padn,), row.dtype)])
    colp = jnp.concatenate([col, jnp.full((padn,), n, col.dtype)])
    ewp = jnp.concatenate([edge_weight,
                           jnp.zeros((padn,), edge_weight.dtype)])
    col2d = colp.reshape(-1, 128)
    ew2d = ewp.reshape(-1, 128)

    degp = _sc_deg(col2d, ew2d)
    d0 = degp[0, :n]
    d1 = degp[1, :n]

    xws, dinv = _tc_a(xv, d0, d1, wtT, bt, wgT)

    acc = _sc_acc(rowp.reshape(-1, 128), col2d, ew2d, xws)

    return _tc_b(acc, xws, dinv, bg, whT, bh)


# speedup vs baseline: 1.0610x; 1.0471x over previous
import functools
import jax
import jax.numpy as jnp
from jax import lax
from jax.experimental import pallas as pl
from jax.experimental.pallas import tpu as pltpu
from jax.experimental.pallas import tpu_sc as plsc

N_NODES = 100000
NDEG = 100352
DEG_SLICE = NDEG // 16


BN = 2048


def _tc_a_body(x_ref, d0_ref, d1_ref, wtT_ref, bt_ref, wgT_ref,
               xws_ref, dinv_ref):
    xv = x_ref[...]
    h = jnp.dot(xv, wtT_ref[...], preferred_element_type=jnp.float32)
    h = jnp.maximum(h + bt_ref[...][None, :], 0.0)
    xw = jnp.dot(h, wgT_ref[...], preferred_element_type=jnp.float32)
    deg = d0_ref[...] + d1_ref[...] + 1.0
    dinv = jnp.where(deg > 0, jax.lax.rsqrt(deg), 0.0)
    dinv_ref[...] = dinv
    xws_ref[...] = xw * dinv[:, None]


def _tc_b_body(acc_ref, xws_ref, dinv_ref, bg_ref, whT_ref, bh_ref, out_ref):
    dinv = dinv_ref[...]
    h2 = dinv[:, None] * (acc_ref[...] + xws_ref[...]) + bg_ref[...][None, :]
    h2 = jnp.maximum(h2, 0.0)
    out_ref[...] = (jnp.sum(h2 * whT_ref[...], axis=1, keepdims=True)
                    + bh_ref[0])


def _sc_deg_body(col2d, ew2d, degp, deg_sp, cv0, cv1, wv0, wv1, zv,
                 ls0, ls1, ss0, ss1):
    c = lax.axis_index("c")
    t = lax.axis_index("s")
    nrows = col2d.shape[0]
    rows_per_core = nrows // 2
    rows_per_tile = rows_per_core // 16
    nchunks = rows_per_tile // 7

    def _z(i, _):
        zv[pl.ds(i * 16, 16)] = jnp.zeros((16,), jnp.float32)
        return 0
    lax.fori_loop(0, DEG_SLICE // 16, _z, 0)
    pltpu.sync_copy(zv, deg_sp.at[pl.ds(t * DEG_SLICE, DEG_SLICE)])
    plsc.subcore_barrier()

    row_base = c * rows_per_core + t * rows_per_tile
    cvs = (cv0, cv1)
    wvs = (wv0, wv1)
    lsems = (ls0, ls1)
    ssems = (ss0, ss1)

    def _stage(k, p):
        r0 = row_base + k * 7
        pltpu.async_copy(col2d.at[pl.ds(r0, 7)], cvs[p], lsems[p])
        pltpu.async_copy(ew2d.at[pl.ds(r0, 7)], wvs[p], lsems[p])

    def _drain(p):
        for j in range(7):
            pltpu.make_async_copy(wvs[p].at[j], deg_sp.at[cvs[p].at[j]],
                                  ssems[p]).wait()

    def _chunk(k, p):
        pltpu.make_async_copy(col2d.at[pl.ds(row_base, 7)],
                              cvs[p], lsems[p]).wait()
        pltpu.make_async_copy(ew2d.at[pl.ds(row_base, 7)],
                              wvs[p], lsems[p]).wait()

        @pl.when(k >= 1)
        def _dr():
            _drain(1 - p)

        @pl.when(k < nchunks - 1)
        def _pf():
            _stage(k + 1, 1 - p)

        for j in range(7):
            pltpu.async_copy(wvs[p].at[j], deg_sp.at[cvs[p].at[j]],
                             ssems[p], add=True)

    _stage(0, 0)

    def _pair(h, _):
        _chunk(2 * h, 0)
        _chunk(2 * h + 1, 1)
        return 0
    lax.fori_loop(0, nchunks // 2, _pair, 0)
    _drain(1)

    plsc.subcore_barrier()
    pltpu.sync_copy(deg_sp.at[pl.ds(t * DEG_SLICE, DEG_SLICE)],
                    degp.at[c].at[pl.ds(t * DEG_SLICE, DEG_SLICE)])


def _sc_deg(col2d, ew2d):
    mesh = plsc.VectorSubcoreMesh(core_axis_name="c", subcore_axis_name="s")
    f = pl.kernel(
        _sc_deg_body,
        out_type=jax.ShapeDtypeStruct((2, NDEG), jnp.float32),
        mesh=mesh,
        scratch_types=[
            pltpu.VMEM_SHARED((NDEG,), jnp.float32),
            pltpu.VMEM((7, 128), jnp.int32),
            pltpu.VMEM((7, 128), jnp.int32),
            pltpu.VMEM((7, 128), jnp.float32),
            pltpu.VMEM((7, 128), jnp.float32),
            pltpu.VMEM((DEG_SLICE,), jnp.float32),
            pltpu.SemaphoreType.DMA,
            pltpu.SemaphoreType.DMA,
            pltpu.SemaphoreType.DMA,
            pltpu.SemaphoreType.DMA,
        ],
        compiler_params=pltpu.CompilerParams(use_tc_tiling_on_sc=False),
    )
    return f(col2d, ew2d)


HALF = 50000
ACC_TILE = 3136
ACC_ROWS = 50304
DUMMY = 50176
OUT_TILE = 3128
OUT_LAST = 50000 - 15 * OUT_TILE


def _sc_acc_body(row2d, col2d, ew2d, xws, acc_out,
                 acc_sp, eb0, eb1, cl0, cl1, wb0, wb1, ix0, ix1,
                 rv0, rv1, rv2, rv3,
                 zb, ls0, ls1, gs0, gs1, gs2, gs3, ss0, ss1, ss2, ss3):
    c = lax.axis_index("c")
    t = lax.axis_index("s")
    base = c * HALF
    nrows = row2d.shape[0]
    rows_per_tile = nrows // 16
    row0 = t * rows_per_tile

    def _z(i, _):
        zb[i, pl.ds(0, 16)] = jnp.zeros((16,), jnp.float32)
        zb[i, pl.ds(16, 16)] = jnp.zeros((16,), jnp.float32)
        return 0
    lax.fori_loop(0, zb.shape[0], _z, 0)
    for m in range(ACC_TILE // 98):
        pltpu.sync_copy(zb, acc_sp.at[pl.ds(t * ACC_TILE + m * 98, 98), :])
    plsc.subcore_barrier()

    ebufs = (eb0, eb1)
    clbufs = (cl0, cl1)
    wbufs = (wb0, wb1)
    ixbufs = (ix0, ix1)
    rvs = (rv0, rv1, rv2, rv3)
    lsems = (ls0, ls1)
    gsems = (gs0, gs1, gs2, gs3)
    ssems = (ss0, ss1, ss2, ss3)
    NG = rows_per_tile // 4

    def _stage(g, p):
        r0 = row0 + g * 4
        pltpu.async_copy(row2d.at[pl.ds(r0, 4)], ebufs[p], lsems[p])
        pltpu.async_copy(col2d.at[pl.ds(r0, 4)], clbufs[p], lsems[p])
        pltpu.async_copy(ew2d.at[pl.ds(r0, 4)], wbufs[p], lsems[p])

    def _wait_stage(p):
        pltpu.make_async_copy(row2d.at[pl.ds(row0, 4)],
                              ebufs[p], lsems[p]).wait()
        pltpu.make_async_copy(col2d.at[pl.ds(row0, 4)],
                              clbufs[p], lsems[p]).wait()
        pltpu.make_async_copy(ew2d.at[pl.ds(row0, 4)],
                              wbufs[p], lsems[p]).wait()

    def _fire_gather(p, j, b):
        pltpu.async_copy(xws.at[ebufs[p].at[j]], rvs[b], gsems[b])

    def _drain_scatter(b, p, j):
        pltpu.make_async_copy(rvs[b], acc_sp.at[ixbufs[p].at[j]],
                              ssems[b]).wait()

    def _group(g, p):
        @pl.when(g < NG - 1)
        def _prefetch():
            _stage(g + 1, 1 - p)

        eb = ebufs[p]
        ix = ixbufs[p]
        for j in range(4):
            jp = (j + 3) % 4
            pltpu.make_async_copy(xws.at[eb.at[j]],
                                  rvs[j], gsems[j]).wait()
            if j >= 1:
                _drain_scatter(jp, p, j - 1)
            else:
                @pl.when(g >= 1)
                def _dr():
                    _drain_scatter(3, 1 - p, 3)
            if j == 0:
                _fire_gather(p, 3, 3)
            else:
                @pl.when(g < NG - 1)
                def _cross():
                    if j == 1:
                        _wait_stage(1 - p)
                    _fire_gather(1 - p, j - 1, j - 1)

            def _scale(k, _):
                q0 = pl.multiple_of(k * 16, 16)
                colg = clbufs[p][j, pl.ds(q0, 16)]
                ewg = wbufs[p][j, pl.ds(q0, 16)]
                tgt = colg - base
                valid = (tgt >= 0) & (tgt < HALF)
                ew_eff = jnp.where(valid, ewg, 0.0)
                spread = DUMMY + q0 + lax.iota(jnp.int32, 16)
                idxg = jnp.where(valid, tgt, spread)
                ix[j, pl.ds(q0, 16)] = idxg
                for u in range(16):
                    s_u = lax.squeeze(lax.slice(ew_eff, (u,), (u + 1,)),
                                      (0,))
                    rvs[j][q0 + u, pl.ds(0, 16)] = (
                        rvs[j][q0 + u, pl.ds(0, 16)] * s_u)
                    rvs[j][q0 + u, pl.ds(16, 16)] = (
                        rvs[j][q0 + u, pl.ds(16, 16)] * s_u)
                return 0
            lax.fori_loop(0, 8, _scale, 0)

            pltpu.async_copy(rvs[j], acc_sp.at[ix.at[j]], ssems[j],
                             add=True)

    _stage(0, 0)
    _wait_stage(0)
    _fire_gather(0, 0, 0)
    _fire_gather(0, 1, 1)
    _fire_gather(0, 2, 2)

    def _pair(h, _):
        _group(2 * h, 0)
        _group(2 * h + 1, 1)
        return 0
    lax.fori_loop(0, NG // 2, _pair, 0)

    _drain_scatter(3, 1, 3)

    plsc.subcore_barrier()

    @pl.when(t < 15)
    def _copy_main():
        pltpu.sync_copy(acc_sp.at[pl.ds(t * OUT_TILE, OUT_TILE), :],
                        acc_out.at[pl.ds(base + t * OUT_TILE, OUT_TILE), :])

    @pl.when(t == 15)
    def _copy_last():
        pltpu.sync_copy(acc_sp.at[pl.ds(15 * OUT_TILE, OUT_LAST), :],
                        acc_out.at[pl.ds(base + 15 * OUT_TILE, OUT_LAST), :])


def _sc_acc(row2d, col2d, ew2d, xws):
    mesh = plsc.VectorSubcoreMesh(core_axis_name="c", subcore_axis_name="s")
    f = pl.kernel(
        _sc_acc_body,
        out_type=jax.ShapeDtypeStruct((N_NODES, 32), jnp.float32),
        mesh=mesh,
        scratch_types=[
            pltpu.VMEM_SHARED((ACC_ROWS, 32), jnp.float32),
            pltpu.VMEM((4, 128), jnp.int32),
            pltpu.VMEM((4, 128), jnp.int32),
            pltpu.VMEM((4, 128), jnp.int32),
            pltpu.VMEM((4, 128), jnp.int32),
            pltpu.VMEM((4, 128), jnp.float32),
            pltpu.VMEM((4, 128), jnp.float32),
            pltpu.VMEM((4, 128), jnp.int32),
            pltpu.VMEM((4, 128), jnp.int32),
            pltpu.VMEM((128, 32), jnp.float32),
            pltpu.VMEM((128, 32), jnp.float32),
            pltpu.VMEM((128, 32), jnp.float32),
            pltpu.VMEM((128, 32), jnp.float32),
            pltpu.VMEM((98, 32), jnp.float32),
            pltpu.SemaphoreType.DMA,
            pltpu.SemaphoreType.DMA,
            pltpu.SemaphoreType.DMA,
            pltpu.SemaphoreType.DMA,
            pltpu.SemaphoreType.DMA,
            pltpu.SemaphoreType.DMA,
            pltpu.SemaphoreType.DMA,
            pltpu.SemaphoreType.DMA,
            pltpu.SemaphoreType.DMA,
            pltpu.SemaphoreType.DMA,
        ],
        compiler_params=pltpu.CompilerParams(use_tc_tiling_on_sc=False),
    )
    return f(row2d, col2d, ew2d, xws)


def _full1d(shape):
    return pl.BlockSpec(shape, lambda i: tuple(0 for _ in shape))


def _tc_a(xv, d0, d1, wtT, bt, wgT):
    n = xv.shape[0]
    grid = pl.cdiv(n, BN)
    return pl.pallas_call(
        _tc_a_body,
        grid=(grid,),
        in_specs=[
            pl.BlockSpec((BN, xv.shape[1]), lambda i: (i, 0)),
            pl.BlockSpec((BN,), lambda i: (i,)),
            pl.BlockSpec((BN,), lambda i: (i,)),
            _full1d(wtT.shape),
            _full1d(bt.shape),
            _full1d(wgT.shape),
        ],
        out_specs=[
            pl.BlockSpec((BN, 32), lambda i: (i, 0)),
            pl.BlockSpec((BN,), lambda i: (i,)),
        ],
        out_shape=[
            jax.ShapeDtypeStruct((n, 32), jnp.float32),
            jax.ShapeDtypeStruct((n,), jnp.float32),
        ],
    )(xv, d0, d1, wtT, bt, wgT)


def _tc_b(acc, xws, dinv, bg, whT, bh):
    n = acc.shape[0]
    grid = pl.cdiv(n, BN)
    return pl.pallas_call(
        _tc_b_body,
        grid=(grid,),
        in_specs=[
            pl.BlockSpec((BN, 32), lambda i: (i, 0)),
            pl.BlockSpec((BN, 32), lambda i: (i, 0)),
            pl.BlockSpec((BN,), lambda i: (i,)),
            _full1d(bg.shape),
            _full1d(whT.shape),
            _full1d(bh.shape),
        ],
        out_specs=pl.BlockSpec((BN, 1), lambda i: (i, 0)),
        out_shape=jax.ShapeDtypeStruct((n, 1), jnp.float32),
    )(acc, xws, dinv, bg, whT, bh)


@jax.jit
def kernel(x, edge_index, edge_weight, Wt, bt, Wg, bg, Wh, bh):
    n = x.shape[0]
    xv = x.reshape(n, -1)
    wtT = Wt.reshape(Wt.shape[0], -1).T
    wgT = Wg.T
    whT = Wh.T
    row = edge_index[0]
    col = edge_index[1]

    e = row.shape[0]
    epad = ((e + 32767) // 32768) * 32768
    padn = epad - e
    rowp = jnp.concatenate([row, jnp.zeros((padn,), row.dtype)])
    colp = jnp.concatenate([col, jnp.full((padn,), n, col.dtype)])
    ewp = jnp.concatenate([edge_weight,
                           jnp.zeros((padn,), edge_weight.dtype)])
    col2d = colp.reshape(-1, 128)
    ew2d = ewp.reshape(-1, 128)

    degp = _sc_deg(col2d, ew2d)
    d0 = degp[0, :n]
    d1 = degp[1, :n]

    xws, dinv = _tc_a(xv, d0, d1, wtT, bt, wgT)

    acc = _sc_acc(rowp.reshape(-1, 128), col2d, ew2d, xws)

    return _tc_b(acc, xws, dinv, bg, whT, bh)
